# baseline (speedup 1.0000x reference)
import jax
import jax.numpy as jnp
from jax.experimental import pallas as pl
from jax.experimental.pallas import tpu as pltpu

_BMA = 400
_BMB = 800


def _a_body(x_ref, w1_ref, adj_ref, b1_ref, w2_ref,
            h1_ref, adjb_ref, z2_ref, z1_scr):
    k = pl.program_id(0)

    @pl.when(k == 0)
    def _():
        z1_scr[...] = jnp.dot(
            x_ref[...], w1_ref[...], preferred_element_type=jnp.float32
        ).astype(jnp.bfloat16)

    ab = adj_ref[...].astype(jnp.bfloat16)
    adjb_ref[...] = ab
    h1 = jnp.dot(ab, z1_scr[...], preferred_element_type=jnp.float32)
    h1 = h1 + b1_ref[...]
    h1_ref[...] = h1
    z2_ref[...] = jnp.dot(
        jnp.maximum(h1, 0.0).astype(jnp.bfloat16), w2_ref[...],
        preferred_element_type=jnp.float32,
    ).astype(jnp.bfloat16)


def _b_body(adjb_ref, z2_ref, b2_ref, b3_ref, w3_ref,
            h2_ref, h3_ref, out_ref, z3_scr):
    l = pl.program_id(0)
    k = pl.program_id(1)

    @pl.when(l == 0)
    def _():
        h2 = jnp.dot(adjb_ref[...], z2_ref[...], preferred_element_type=jnp.float32)
        h2 = h2 + b2_ref[...]
        h2_ref[...] = h2
        z3_scr[pl.ds(k * _BMB, _BMB), :] = jnp.dot(
            jnp.maximum(h2, 0.0).astype(jnp.bfloat16), w3_ref[...],
            preferred_element_type=jnp.float32,
        ).astype(jnp.bfloat16)

    @pl.when(l == 1)
    def _():
        h3 = jnp.dot(adjb_ref[...], z3_scr[pl.ds(0, 10000), :], preferred_element_type=jnp.float32)
        h3 = h3 + b3_ref[...]
        h3_ref[...] = h3
        m = jnp.max(h3, axis=1, keepdims=True)
        lse = jnp.log(jnp.sum(jnp.exp(h3 - m), axis=1, keepdims=True)) + m
        out_ref[...] = h3 - lse


def kernel(x, adj, W1, b1, W2, b2, W3, b3):
    n, nfeat = x.shape
    nhid = W1.shape[1]
    ncls = W3.shape[1]

    h1, adjb, z2 = pl.pallas_call(
        _a_body,
        grid=(n // _BMA,),
        in_specs=[
            pl.BlockSpec((n, nfeat), lambda i: (0, 0)),
            pl.BlockSpec((nfeat, nhid), lambda i: (0, 0)),
            pl.BlockSpec((_BMA, n), lambda i: (i, 0)),
            pl.BlockSpec((1, nhid), lambda i: (0, 0)),
            pl.BlockSpec((nhid, nhid), lambda i: (0, 0)),
        ],
        out_specs=[
            pl.BlockSpec((_BMA, nhid), lambda i: (i, 0)),
            pl.BlockSpec((_BMA, n), lambda i: (i, 0)),
            pl.BlockSpec((_BMA, nhid), lambda i: (i, 0)),
        ],
        out_shape=[
            jax.ShapeDtypeStruct((n, nhid), jnp.float32),
            jax.ShapeDtypeStruct((n, n), jnp.bfloat16),
            jax.ShapeDtypeStruct((n, nhid), jnp.bfloat16),
        ],
        scratch_shapes=[pltpu.VMEM((n, nhid), jnp.bfloat16)],
        compiler_params=pltpu.CompilerParams(vmem_limit_bytes=67108864),
    )(x, W1, adj, b1.reshape(1, nhid), W2.astype(jnp.bfloat16))

    h2, h3, out = pl.pallas_call(
        _b_body,
        grid=(2, pl.cdiv(n, _BMB)),
        in_specs=[
            pl.BlockSpec((_BMB, n), lambda l, k: (k, 0)),
            pl.BlockSpec((n, nhid), lambda l, k: (0, 0)),
            pl.BlockSpec((1, nhid), lambda l, k: (0, 0)),
            pl.BlockSpec((1, ncls), lambda l, k: (0, 0)),
            pl.BlockSpec((nhid, ncls), lambda l, k: (0, 0)),
        ],
        out_specs=[
            pl.BlockSpec((_BMB, nhid), lambda l, k: (jnp.where(l == 0, k, pl.cdiv(n, _BMB) - 1), 0)),
            pl.BlockSpec((_BMB, ncls), lambda l, k: (jnp.where(l == 1, k, 0), 0)),
            pl.BlockSpec((_BMB, ncls), lambda l, k: (jnp.where(l == 1, k, 0), 0)),
        ],
        out_shape=[
            jax.ShapeDtypeStruct((n, nhid), jnp.float32),
            jax.ShapeDtypeStruct((n, ncls), jnp.float32),
            jax.ShapeDtypeStruct((n, ncls), jnp.float32),
        ],
        scratch_shapes=[pltpu.VMEM((pl.cdiv(n, _BMB) * _BMB, ncls), jnp.bfloat16)],
        compiler_params=pltpu.CompilerParams(
            dimension_semantics=("arbitrary", "arbitrary"),
            vmem_limit_bytes=67108864,
        ),
    )(adjb, z2, b2.reshape(1, nhid), b3.reshape(1, ncls), W3.astype(jnp.bfloat16))

    return (out, h1, h2, h3)


# speedup vs baseline: 1.0179x; 1.0179x over previous
import jax
import jax.numpy as jnp
from jax.experimental import pallas as pl
from jax.experimental.pallas import tpu as pltpu

_BMA = 400
_BMB = 1000


def _a_body(x_ref, w1_ref, adj_ref, b1_ref, w2_ref,
            h1_ref, adjb_ref, z2_ref, z1_scr):
    k = pl.program_id(0)

    @pl.when(k == 0)
    def _():
        z1_scr[...] = jnp.dot(
            x_ref[...], w1_ref[...], preferred_element_type=jnp.float32
        ).astype(jnp.bfloat16)

    ab = adj_ref[...].astype(jnp.bfloat16)
    adjb_ref[...] = ab
    h1 = jnp.dot(ab, z1_scr[...], preferred_element_type=jnp.float32)
    h1 = h1 + b1_ref[...]
    h1_ref[...] = h1
    z2_ref[...] = jnp.dot(
        jnp.maximum(h1, 0.0).astype(jnp.bfloat16), w2_ref[...],
        preferred_element_type=jnp.float32,
    ).astype(jnp.bfloat16)


def _b_body(adjb_ref, z2_ref, b2_ref, b3_ref, w3_ref,
            h2_ref, h3_ref, out_ref, z3_scr):
    l = pl.program_id(0)
    k = pl.program_id(1)

    @pl.when(l == 0)
    def _():
        h2 = jnp.dot(adjb_ref[...], z2_ref[...], preferred_element_type=jnp.float32)
        h2 = h2 + b2_ref[...]
        h2_ref[...] = h2
        z3_scr[pl.ds(k * _BMB, _BMB), :] = jnp.dot(
            jnp.maximum(h2, 0.0).astype(jnp.bfloat16), w3_ref[...],
            preferred_element_type=jnp.float32,
        )

    @pl.when(l == 1)
    def _():
        h3 = jnp.dot(adjb_ref[...], z3_scr[...].astype(jnp.bfloat16), preferred_element_type=jnp.float32)
        h3 = h3 + b3_ref[...]
        h3_ref[...] = h3
        m = jnp.max(h3, axis=1, keepdims=True)
        lse = jnp.log(jnp.sum(jnp.exp(h3 - m), axis=1, keepdims=True)) + m
        out_ref[...] = h3 - lse


def kernel(x, adj, W1, b1, W2, b2, W3, b3):
    n, nfeat = x.shape
    nhid = W1.shape[1]
    ncls = W3.shape[1]

    h1, adjb, z2 = pl.pallas_call(
        _a_body,
        grid=(n // _BMA,),
        in_specs=[
            pl.BlockSpec((n, nfeat), lambda i: (0, 0)),
            pl.BlockSpec((nfeat, nhid), lambda i: (0, 0)),
            pl.BlockSpec((_BMA, n), lambda i: (i, 0)),
            pl.BlockSpec((1, nhid), lambda i: (0, 0)),
            pl.BlockSpec((nhid, nhid), lambda i: (0, 0)),
        ],
        out_specs=[
            pl.BlockSpec((_BMA, nhid), lambda i: (i, 0)),
            pl.BlockSpec((_BMA, n), lambda i: (i, 0)),
            pl.BlockSpec((_BMA, nhid), lambda i: (i, 0)),
        ],
        out_shape=[
            jax.ShapeDtypeStruct((n, nhid), jnp.float32),
            jax.ShapeDtypeStruct((n, n), jnp.bfloat16),
            jax.ShapeDtypeStruct((n, nhid), jnp.bfloat16),
        ],
        scratch_shapes=[pltpu.VMEM((n, nhid), jnp.bfloat16)],
        compiler_params=pltpu.CompilerParams(vmem_limit_bytes=67108864),
    )(x, W1, adj, b1.reshape(1, nhid), W2.astype(jnp.bfloat16))

    h2, h3, out = pl.pallas_call(
        _b_body,
        grid=(2, n // _BMB),
        in_specs=[
            pl.BlockSpec((_BMB, n), lambda l, k: (k, 0)),
            pl.BlockSpec((n, nhid), lambda l, k: (0, 0)),
            pl.BlockSpec((1, nhid), lambda l, k: (0, 0)),
            pl.BlockSpec((1, ncls), lambda l, k: (0, 0)),
            pl.BlockSpec((nhid, ncls), lambda l, k: (0, 0)),
        ],
        out_specs=[
            pl.BlockSpec((_BMB, nhid), lambda l, k: (jnp.where(l == 0, k, n // _BMB - 1), 0)),
            pl.BlockSpec((_BMB, ncls), lambda l, k: (jnp.where(l == 1, k, 0), 0)),
            pl.BlockSpec((_BMB, ncls), lambda l, k: (jnp.where(l == 1, k, 0), 0)),
        ],
        out_shape=[
            jax.ShapeDtypeStruct((n, nhid), jnp.float32),
            jax.ShapeDtypeStruct((n, ncls), jnp.float32),
            jax.ShapeDtypeStruct((n, ncls), jnp.float32),
        ],
        scratch_shapes=[pltpu.VMEM((n, ncls), jnp.float32)],
        compiler_params=pltpu.CompilerParams(
            dimension_semantics=("arbitrary", "arbitrary"),
            vmem_limit_bytes=67108864,
        ),
    )(adjb, z2, b2.reshape(1, nhid), b3.reshape(1, ncls), W3.astype(jnp.bfloat16))

    return (out, h1, h2, h3)
